# baseline (speedup 1.0000x reference)
import functools

import jax
import jax.numpy as jnp
from jax import lax
from jax.experimental import pallas as pl
from jax.experimental.pallas import tpu as pltpu
from jax.experimental.pallas import tpu_sc as plsc

_info = plsc.get_sparse_core_info()
_NC, _NS = _info.num_cores, _info.num_subcores
_NW = _NC * _NS
_C = 128
_NBUF = 4
_AHEAD = 2


@functools.partial(jax.jit, static_argnames=("n",))
def _gather(table, idx, n):
    d = table.shape[1]
    n_chunks = -(-n // (_NW * _C))
    per_w = n_chunks * _C
    full_w = n // per_w
    rem = n - full_w * per_w
    fc, tr = rem // _C, rem % _C
    mesh = plsc.VectorSubcoreMesh(core_axis_name="c", subcore_axis_name="s")

    @functools.partial(
        pl.kernel,
        mesh=mesh,
        out_type=jax.ShapeDtypeStruct((n, d), jnp.float32),
        scratch_types=[
            pltpu.VMEM_SHARED(table.shape, jnp.float32),
            pltpu.VMEM((per_w,), jnp.int32),
            pltpu.VMEM((_C, d), jnp.float32),
            pltpu.VMEM((_C, d), jnp.float32),
            pltpu.VMEM((_C, d), jnp.float32),
            pltpu.VMEM((_C, d), jnp.float32),
            pltpu.SemaphoreType.DMA,
            pltpu.SemaphoreType.DMA,
            pltpu.SemaphoreType.DMA,
            pltpu.SemaphoreType.DMA,
            pltpu.SemaphoreType.DMA,
            pltpu.SemaphoreType.DMA,
            pltpu.SemaphoreType.DMA,
            pltpu.SemaphoreType.DMA,
        ],
    )
    def k(table_hbm, idx_hbm, out_hbm, table_v, idx_v,
          buf0, buf1, buf2, buf3, gs0, gs1, gs2, gs3, ws0, ws1, ws2, ws3):
        sid = lax.axis_index("s")
        wid = sid * _NC + lax.axis_index("c")
        base = wid * per_w

        @pl.when(sid == 0)
        def _copy_table():
            pltpu.sync_copy(table_hbm, table_v)

        @pl.when(wid < full_w)
        def _copy_idx_full():
            pltpu.sync_copy(idx_hbm.at[pl.ds(base, per_w)], idx_v)

        if rem > 0:
            @pl.when(wid == full_w)
            def _copy_idx_partial():
                pltpu.sync_copy(
                    idx_hbm.at[pl.ds(base, rem)], idx_v.at[pl.ds(0, rem)]
                )

        plsc.subcore_barrier()

        bufs = (buf0, buf1, buf2, buf3)
        gsems = (gs0, gs1, gs2, gs3)
        wsems = (ws0, ws1, ws2, ws3)

        def gather_chunk(j, b):
            return pltpu.async_copy(
                table_v.at[idx_v.at[pl.ds(j * _C, _C)]], bufs[b], gsems[b]
            )

        @pl.when(wid < full_w)
        def _full_slab():
            gathers = [None] * n_chunks
            writes = [None] * _NBUF
            for m in range(min(_AHEAD + 1, n_chunks)):
                gathers[m] = gather_chunk(m, m % _NBUF)
            for j in range(n_chunks):
                gathers[j].wait()
                w = pltpu.async_copy(
                    bufs[j % _NBUF],
                    out_hbm.at[pl.ds(base + j * _C, _C)],
                    wsems[j % _NBUF],
                )
                nxt = j + _AHEAD + 1
                if nxt < n_chunks:
                    b = nxt % _NBUF
                    if writes[b] is not None:
                        writes[b].wait()
                    gathers[nxt] = gather_chunk(nxt, b)
                writes[j % _NBUF] = w
            for b in range(_NBUF):
                if writes[b] is not None:
                    writes[b].wait()

        if rem > 0:
            @pl.when(wid == full_w)
            def _partial_slab():
                for j in range(fc):
                    gather_chunk(j, j % _NBUF).wait()
                    pltpu.sync_copy(
                        bufs[j % _NBUF], out_hbm.at[pl.ds(base + j * _C, _C)]
                    )
                if tr > 0:
                    pltpu.async_copy(
                        table_v.at[idx_v.at[pl.ds(fc * _C, tr)]],
                        bufs[fc % _NBUF].at[pl.ds(0, tr)],
                        gsems[fc % _NBUF],
                    ).wait()
                    pltpu.sync_copy(
                        bufs[fc % _NBUF].at[pl.ds(0, tr)],
                        out_hbm.at[pl.ds(base + fc * _C, tr)],
                    )

    return k(table, idx)


def kernel(atomic_numbers, embedding_weight):
    n = atomic_numbers.shape[0]
    idx = atomic_numbers.astype(jnp.int32)
    return _gather(embedding_weight, idx, n)
